# baseline (speedup 1.0000x reference)
import jax
import jax.numpy as jnp
from jax import lax
from jax.experimental import pallas as pl
from jax.experimental.pallas import tpu as pltpu
from jax.experimental.pallas import tpu_sc as plsc

_RADIUS2 = 4.0
_SCORES_TH = 0.1
_W = 639.0
_H = 479.0
_N = 5000
_NW = 16
_NWORK = 32
_NPAD = 5120
_PW = _NPAD // _NW
_PB = _NPAD // _NWORK
_L = 16
_NSTR = 324
_SSZ = 336
_CSZ = 352
_SCHUNK = 80
_NEG = -3.0e38
_PADX = 1.0e6
_NTAIL = _N - (_NW - 1) * _PW


def _sc_body(kh, sh, dh, out_sum, out_cnt,
             kv2, xv, yv, sv, dv, sidv, occv, lastv, posv1, posv2,
             cnt, allcnt, totv, wpartv, Cv,
             sx, sy, ss, sdv, psumr, pcntr, sem,
             shared_cnt, shared_sx, shared_sy, shared_ss, shared_sd):
    wc = lax.axis_index("c")
    wid = lax.axis_index("s")
    rid = wid * 2 + wc
    base = wid * _PW
    lane = lax.iota(jnp.int32, _L)
    nvec = _PW // _L
    ones_i = jnp.ones((_L,), jnp.int32)
    zeros_i = jnp.zeros((_L,), jnp.int32)

    scope_a = jax.named_scope("ph_load")
    scope_a.__enter__()

    @pl.when(wid < _NW - 1)
    def _():
        pltpu.sync_copy(kh.at[pl.ds(base, _PW), :], kv2)
        pltpu.sync_copy(sh.at[pl.ds(base, _PW)], sv)
        pltpu.sync_copy(dh.at[pl.ds(base, _PW)], dv)

    @pl.when(wid == _NW - 1)
    def _():
        pltpu.sync_copy(kh.at[pl.ds(base, _NTAIL), :],
                        kv2.at[pl.ds(0, _NTAIL), :])
        pltpu.sync_copy(sh.at[pl.ds(base, _NTAIL)], sv.at[pl.ds(0, _NTAIL)])
        pltpu.sync_copy(dh.at[pl.ds(base, _NTAIL)], dv.at[pl.ds(0, _NTAIL)])

    col0 = jnp.zeros((_L,), jnp.int32)
    col1 = jnp.ones((_L,), jnp.int32)

    def deint_step(k, c):
        sl = pl.ds(k * _L, _L)
        rows = k * _L + lane
        xv[sl] = plsc.load_gather(kv2, [rows, col0]) * _W
        yv[sl] = plsc.load_gather(kv2, [rows, col1]) * _H
        return c

    lax.fori_loop(0, nvec, deint_step, 0)

    @pl.when(wid == _NW - 1)
    def _():
        for k in range(_NTAIL // _L, nvec):
            sl = pl.ds(k * _L, _L)
            m = k * _L + lane < _NTAIL
            xv[sl] = jnp.where(m, xv[sl], _PADX)
            yv[sl] = jnp.where(m, yv[sl], _PADX)
            sv[sl] = jnp.where(m, sv[sl], -1.0)
            dv[sl] = jnp.where(m, dv[sl], 0.0)

    def sid_step(k, c):
        sl = pl.ds(k * _L, _L)
        sidv[sl] = jnp.minimum((xv[sl] * 0.5).astype(jnp.int32), _NSTR - 1)
        return c

    lax.fori_loop(0, nvec, sid_step, 0)

    for k in range(_SSZ // _L):
        cnt[pl.ds(k * _L, _L)] = zeros_i

    def count_step(k, c):
        sl = pl.ds(k * _L, _L)
        sid = sidv[sl]
        occ, last = plsc.scan_count(sid)
        occv[sl] = occ
        lastv[sl] = last.astype(jnp.int32)
        cur = plsc.load_gather(cnt, [sid])
        plsc.store_scatter(cnt, [sid], cur + occ, mask=last)
        return c

    lax.fori_loop(0, nvec, count_step, 0)
    scope_a.__exit__(None, None, None)

    scope_g = jax.named_scope("ph_agg")
    scope_g.__enter__()
    pltpu.sync_copy(cnt, shared_cnt.at[wid])
    plsc.subcore_barrier()
    pltpu.sync_copy(shared_cnt, allcnt)

    for k in range(_SSZ // _L):
        sl = pl.ds(k * _L, _L)
        tot = jnp.zeros((_L,), jnp.int32)
        part = jnp.zeros((_L,), jnp.int32)
        for w in range(_NW):
            row = allcnt[w, sl]
            tot = tot + row
            part = part + row * (jnp.int32(w) < wid).astype(jnp.int32)
        totv[sl] = tot
        wpartv[sl] = part

    npad_i = jnp.full((_L,), _NPAD, jnp.int32)
    for k in range(_SSZ // _L, _CSZ // _L):
        Cv[pl.ds(k * _L, _L)] = npad_i

    def cum_step(k, carry):
        sl = pl.ds(k * _L, _L)
        v = totv[sl]
        p = plsc.cumsum(v)
        Cv[sl] = p - v + carry
        return carry + p[_L - 1]

    lax.fori_loop(0, _SSZ // _L, cum_step, jnp.int32(0))

    def curs_step(k, c):
        sl = pl.ds(k * _L, _L)
        wpartv[sl] = Cv[sl] + wpartv[sl]
        return c

    lax.fori_loop(0, _SSZ // _L, curs_step, 0)
    scope_g.__exit__(None, None, None)

    scope_p = jax.named_scope("ph_place")
    scope_p.__enter__()
    def place_step(k, c):
        sl = pl.ds(k * _L, _L)
        sid = sidv[sl]
        occ = occv[sl]
        last = lastv[sl] == 1
        b = plsc.load_gather(wpartv, [sid])
        posv1[sl] = b + occ - 1
        plsc.store_scatter(wpartv, [sid], b + occ, mask=last)
        return c

    lax.fori_loop(0, nvec, place_step, 0)

    for k in range(nvec):
        posv2[k // (_SCHUNK // _L),
              pl.ds((k % (_SCHUNK // _L)) * _L, _L)] = posv1[pl.ds(k * _L, _L)]

    descs = []
    for c in range(_PW // _SCHUNK):
        sl = pl.ds(c * _SCHUNK, _SCHUNK)
        idx = posv2.at[c]
        descs.append(pltpu.async_copy(xv.at[sl], shared_sx.at[idx], sem))
        descs.append(pltpu.async_copy(yv.at[sl], shared_sy.at[idx], sem))
        descs.append(pltpu.async_copy(sv.at[sl], shared_ss.at[idx], sem))
        descs.append(pltpu.async_copy(dv.at[sl], shared_sd.at[idx], sem))
    for dsc in descs:
        dsc.wait()
    plsc.subcore_barrier()
    scope_p.__exit__(None, None, None)

    scope_c = jax.named_scope("ph_copyback")
    scope_c.__enter__()
    bbase = rid * _PB
    pltpu.sync_copy(shared_sx, sx)
    pltpu.sync_copy(shared_sy, sy)
    pltpu.sync_copy(shared_ss, ss)
    pltpu.sync_copy(shared_sd.at[pl.ds(bbase, _PB)], sdv)
    scope_c.__exit__(None, None, None)

    scope_n = jax.named_scope("ph_nms")
    scope_n.__enter__()
    psumr[...] = jnp.zeros((_L,), jnp.float32)
    pcntr[...] = jnp.zeros((_L,), jnp.float32)

    def group_step(grp, carry):
        g0 = bbase + grp * _L
        xi16 = sx[pl.ds(g0, _L)]
        yi16 = sy[pl.ds(g0, _L)]
        si16 = ss[pl.ds(g0, _L)]
        di16 = sdv[pl.ds(grp * _L, _L)]
        sid16 = jnp.minimum((xi16 * 0.5).astype(jnp.int32), _NSTR - 1)
        lo16 = plsc.load_gather(Cv, [jnp.maximum(sid16 - 1, 0)])
        hi16 = plsc.load_gather(Cv, [sid16 + 2])
        jb0 = lax.shift_right_logical(lo16[0], 4)
        jb1 = lax.shift_right_logical(hi16[_L - 1] + (_L - 1), 4)

        def cand_step(jb, acc):
            sl = pl.ds(jb * _L, _L)
            xj16 = sx[sl]
            yj16 = sy[sl]
            sj16 = ss[sl]
            for t in range(_L):
                dx = xi16 - xj16[t]
                dy = yi16 - yj16[t]
                d2 = dx * dx + dy * dy
                acc = jnp.maximum(acc,
                                  jnp.where(d2 < _RADIUS2, sj16[t], _NEG))
            return acc

        acc = lax.fori_loop(jb0, jb1, cand_step,
                            jnp.full((_L,), _NEG, jnp.float32))
        valid = jnp.logical_and(si16 >= acc, si16 > _SCORES_TH)
        vf16 = valid.astype(jnp.float32)
        psumr[...] = psumr[...] + vf16 * di16
        pcntr[...] = pcntr[...] + vf16
        return carry

    lax.fori_loop(0, _PB // _L, group_step, 0)
    scope_n.__exit__(None, None, None)

    pltpu.sync_copy(psumr, out_sum.at[rid, pl.ds(0, _L)])
    pltpu.sync_copy(pcntr, out_cnt.at[rid, pl.ds(0, _L)])


def _sc_call(kpts, s, d):
    mesh = plsc.VectorSubcoreMesh(core_axis_name="c", subcore_axis_name="s",
                                  num_cores=2)
    f = pl.kernel(
        _sc_body,
        out_type=(jax.ShapeDtypeStruct((_NWORK, _L), jnp.float32),
                  jax.ShapeDtypeStruct((_NWORK, _L), jnp.float32)),
        mesh=mesh,
        compiler_params=pltpu.CompilerParams(needs_layout_passes=False),
        scratch_types=[
            pltpu.VMEM((_PW, 2), jnp.float32),
            pltpu.VMEM((_PW,), jnp.float32),
            pltpu.VMEM((_PW,), jnp.float32),
            pltpu.VMEM((_PW,), jnp.float32),
            pltpu.VMEM((_PW,), jnp.float32),
            pltpu.VMEM((_PW,), jnp.int32),
            pltpu.VMEM((_PW,), jnp.int32),
            pltpu.VMEM((_PW,), jnp.int32),
            pltpu.VMEM((_PW,), jnp.int32),
            pltpu.VMEM((_PW // _SCHUNK, _SCHUNK), jnp.int32),
            pltpu.VMEM((_SSZ,), jnp.int32),
            pltpu.VMEM((_NW, _SSZ), jnp.int32),
            pltpu.VMEM((_SSZ,), jnp.int32),
            pltpu.VMEM((_SSZ,), jnp.int32),
            pltpu.VMEM((_CSZ,), jnp.int32),
            pltpu.VMEM((_NPAD,), jnp.float32),
            pltpu.VMEM((_NPAD,), jnp.float32),
            pltpu.VMEM((_NPAD,), jnp.float32),
            pltpu.VMEM((_PB,), jnp.float32),
            pltpu.VMEM((_L,), jnp.float32),
            pltpu.VMEM((_L,), jnp.float32),
            pltpu.SemaphoreType.DMA,
            pltpu.VMEM_SHARED((_NW, _SSZ), jnp.int32),
            pltpu.VMEM_SHARED((_NPAD,), jnp.float32),
            pltpu.VMEM_SHARED((_NPAD,), jnp.float32),
            pltpu.VMEM_SHARED((_NPAD,), jnp.float32),
            pltpu.VMEM_SHARED((_NPAD,), jnp.float32),
        ],
    )
    return f(kpts, s, d)


def kernel(kpts, scores, dispersity):
    out_sum, out_cnt = _sc_call(kpts, scores, dispersity)
    loss_sum = jnp.sum(out_sum)
    cnt = jnp.sum(out_cnt)
    return jnp.where(cnt > 0, loss_sum / jnp.maximum(cnt, 1.0),
                     jnp.float32(0.0))


# speedup vs baseline: 1.0831x; 1.0831x over previous
import jax
import jax.numpy as jnp
from jax import lax
from jax.experimental import pallas as pl
from jax.experimental.pallas import tpu as pltpu
from jax.experimental.pallas import tpu_sc as plsc

_RADIUS2 = 4.0
_SCORES_TH = 0.1
_W = 639.0
_H = 479.0
_N = 5000
_NW = 16
_NWORK = 32
_NPAD = 5120
_PW = _NPAD // _NW
_PB = _NPAD // _NWORK
_L = 16
_NSTR = 324
_SSZ = 336
_CSZ = 352
_SCHUNK = 80
_NEG = -3.0e38
_PADX = 1.0e6


def _sc_body(xh, yh, sh, dh, out_sum, out_cnt,
             xv, yv, sv, dv, sidv, occv, lastv, posv1, posv2,
             cnt, allcnt, totv, wpartv, Cv,
             sx, sy, ss, sdv, psumr, pcntr, sem,
             shared_cnt, shared_sx, shared_sy, shared_ss, shared_sd):
    wc = lax.axis_index("c")
    wid = lax.axis_index("s")
    rid = wid * 2 + wc
    base = wid * _PW
    lane = lax.iota(jnp.int32, _L)
    nvec = _PW // _L
    ones_i = jnp.ones((_L,), jnp.int32)
    zeros_i = jnp.zeros((_L,), jnp.int32)

    scope_a = jax.named_scope("ph_load")
    scope_a.__enter__()
    pltpu.sync_copy(xh.at[pl.ds(base, _PW)], xv)
    pltpu.sync_copy(yh.at[pl.ds(base, _PW)], yv)
    pltpu.sync_copy(sh.at[pl.ds(base, _PW)], sv)
    pltpu.sync_copy(dh.at[pl.ds(base, _PW)], dv)

    def sid_step(k, c):
        sl = pl.ds(k * _L, _L)
        sidv[sl] = jnp.minimum((xv[sl] * 0.5).astype(jnp.int32), _NSTR - 1)
        return c

    lax.fori_loop(0, nvec, sid_step, 0)

    for k in range(_SSZ // _L):
        cnt[pl.ds(k * _L, _L)] = zeros_i

    def count_step(k, c):
        sl = pl.ds(k * _L, _L)
        sid = sidv[sl]
        occ, last = plsc.scan_count(sid)
        occv[sl] = occ
        lastv[sl] = last.astype(jnp.int32)
        cur = plsc.load_gather(cnt, [sid])
        plsc.store_scatter(cnt, [sid], cur + occ, mask=last)
        return c

    lax.fori_loop(0, nvec, count_step, 0)
    scope_a.__exit__(None, None, None)

    scope_g = jax.named_scope("ph_agg")
    scope_g.__enter__()
    pltpu.sync_copy(cnt, shared_cnt.at[wid])
    plsc.subcore_barrier()
    pltpu.sync_copy(shared_cnt, allcnt)

    for k in range(_SSZ // _L):
        sl = pl.ds(k * _L, _L)
        tot = jnp.zeros((_L,), jnp.int32)
        part = jnp.zeros((_L,), jnp.int32)
        for w in range(_NW):
            row = allcnt[w, sl]
            tot = tot + row
            part = part + row * (jnp.int32(w) < wid).astype(jnp.int32)
        totv[sl] = tot
        wpartv[sl] = part

    npad_i = jnp.full((_L,), _NPAD, jnp.int32)
    for k in range(_SSZ // _L, _CSZ // _L):
        Cv[pl.ds(k * _L, _L)] = npad_i

    def cum_step(k, carry):
        sl = pl.ds(k * _L, _L)
        v = totv[sl]
        p = plsc.cumsum(v)
        Cv[sl] = p - v + carry
        return carry + p[_L - 1]

    lax.fori_loop(0, _SSZ // _L, cum_step, jnp.int32(0))

    def curs_step(k, c):
        sl = pl.ds(k * _L, _L)
        wpartv[sl] = Cv[sl] + wpartv[sl]
        return c

    lax.fori_loop(0, _SSZ // _L, curs_step, 0)
    scope_g.__exit__(None, None, None)

    scope_p = jax.named_scope("ph_place")
    scope_p.__enter__()
    def place_step(k, c):
        sl = pl.ds(k * _L, _L)
        sid = sidv[sl]
        occ = occv[sl]
        last = lastv[sl] == 1
        b = plsc.load_gather(wpartv, [sid])
        posv1[sl] = b + occ - 1
        plsc.store_scatter(wpartv, [sid], b + occ, mask=last)
        return c

    lax.fori_loop(0, nvec, place_step, 0)

    for k in range(nvec):
        posv2[k // (_SCHUNK // _L),
              pl.ds((k % (_SCHUNK // _L)) * _L, _L)] = posv1[pl.ds(k * _L, _L)]

    descs = []
    for c in range(_PW // _SCHUNK):
        sl = pl.ds(c * _SCHUNK, _SCHUNK)
        idx = posv2.at[c]
        descs.append(pltpu.async_copy(xv.at[sl], shared_sx.at[idx], sem))
        descs.append(pltpu.async_copy(yv.at[sl], shared_sy.at[idx], sem))
        descs.append(pltpu.async_copy(sv.at[sl], shared_ss.at[idx], sem))
        descs.append(pltpu.async_copy(dv.at[sl], shared_sd.at[idx], sem))
    for dsc in descs:
        dsc.wait()
    plsc.subcore_barrier()
    scope_p.__exit__(None, None, None)

    scope_c = jax.named_scope("ph_copyback")
    scope_c.__enter__()
    bbase = rid * _PB
    pltpu.sync_copy(shared_sx, sx)
    pltpu.sync_copy(shared_sy, sy)
    pltpu.sync_copy(shared_ss, ss)
    pltpu.sync_copy(shared_sd.at[pl.ds(bbase, _PB)], sdv)
    scope_c.__exit__(None, None, None)

    scope_n = jax.named_scope("ph_nms")
    scope_n.__enter__()
    psumr[...] = jnp.zeros((_L,), jnp.float32)
    pcntr[...] = jnp.zeros((_L,), jnp.float32)

    def group_step(grp, carry):
        g0 = bbase + grp * _L
        xi16 = sx[pl.ds(g0, _L)]
        yi16 = sy[pl.ds(g0, _L)]
        si16 = ss[pl.ds(g0, _L)]
        di16 = sdv[pl.ds(grp * _L, _L)]
        sid16 = jnp.minimum((xi16 * 0.5).astype(jnp.int32), _NSTR - 1)
        lo16 = plsc.load_gather(Cv, [jnp.maximum(sid16 - 1, 0)])
        hi16 = plsc.load_gather(Cv, [sid16 + 2])
        jb0 = lax.shift_right_logical(lo16[0], 4)
        jb1 = lax.shift_right_logical(hi16[_L - 1] + (_L - 1), 4)

        def cand_step(jb, acc):
            sl = pl.ds(jb * _L, _L)
            xj16 = sx[sl]
            yj16 = sy[sl]
            sj16 = ss[sl]
            for t in range(_L):
                dx = xi16 - xj16[t]
                dy = yi16 - yj16[t]
                d2 = dx * dx + dy * dy
                acc = jnp.maximum(acc,
                                  jnp.where(d2 < _RADIUS2, sj16[t], _NEG))
            return acc

        acc = lax.fori_loop(jb0, jb1, cand_step,
                            jnp.full((_L,), _NEG, jnp.float32))
        valid = jnp.logical_and(si16 >= acc, si16 > _SCORES_TH)
        vf16 = valid.astype(jnp.float32)
        psumr[...] = psumr[...] + vf16 * di16
        pcntr[...] = pcntr[...] + vf16
        return carry

    lax.fori_loop(0, _PB // _L, group_step, 0)
    scope_n.__exit__(None, None, None)

    pltpu.sync_copy(psumr, out_sum.at[rid, pl.ds(0, _L)])
    pltpu.sync_copy(pcntr, out_cnt.at[rid, pl.ds(0, _L)])


def _sc_call(x, y, s, d):
    mesh = plsc.VectorSubcoreMesh(core_axis_name="c", subcore_axis_name="s",
                                  num_cores=2)
    f = pl.kernel(
        _sc_body,
        out_type=(jax.ShapeDtypeStruct((_NWORK, _L), jnp.float32),
                  jax.ShapeDtypeStruct((_NWORK, _L), jnp.float32)),
        mesh=mesh,
        compiler_params=pltpu.CompilerParams(needs_layout_passes=False),
        scratch_types=[
            pltpu.VMEM((_PW,), jnp.float32),
            pltpu.VMEM((_PW,), jnp.float32),
            pltpu.VMEM((_PW,), jnp.float32),
            pltpu.VMEM((_PW,), jnp.float32),
            pltpu.VMEM((_PW,), jnp.int32),
            pltpu.VMEM((_PW,), jnp.int32),
            pltpu.VMEM((_PW,), jnp.int32),
            pltpu.VMEM((_PW,), jnp.int32),
            pltpu.VMEM((_PW // _SCHUNK, _SCHUNK), jnp.int32),
            pltpu.VMEM((_SSZ,), jnp.int32),
            pltpu.VMEM((_NW, _SSZ), jnp.int32),
            pltpu.VMEM((_SSZ,), jnp.int32),
            pltpu.VMEM((_SSZ,), jnp.int32),
            pltpu.VMEM((_CSZ,), jnp.int32),
            pltpu.VMEM((_NPAD,), jnp.float32),
            pltpu.VMEM((_NPAD,), jnp.float32),
            pltpu.VMEM((_NPAD,), jnp.float32),
            pltpu.VMEM((_PB,), jnp.float32),
            pltpu.VMEM((_L,), jnp.float32),
            pltpu.VMEM((_L,), jnp.float32),
            pltpu.SemaphoreType.DMA,
            pltpu.VMEM_SHARED((_NW, _SSZ), jnp.int32),
            pltpu.VMEM_SHARED((_NPAD,), jnp.float32),
            pltpu.VMEM_SHARED((_NPAD,), jnp.float32),
            pltpu.VMEM_SHARED((_NPAD,), jnp.float32),
            pltpu.VMEM_SHARED((_NPAD,), jnp.float32),
        ],
    )
    return f(x, y, s, d)


def kernel(kpts, scores, dispersity):
    x = kpts[:, 0] * _W
    y = kpts[:, 1] * _H
    pad = _NPAD - _N
    x = jnp.concatenate([x, jnp.full((pad,), _PADX, jnp.float32)])
    y = jnp.concatenate([y, jnp.full((pad,), _PADX, jnp.float32)])
    s = jnp.concatenate([scores, jnp.full((pad,), -1.0, jnp.float32)])
    d = jnp.concatenate([dispersity, jnp.zeros((pad,), jnp.float32)])
    out_sum, out_cnt = _sc_call(x, y, s, d)
    loss_sum = jnp.sum(out_sum)
    cnt = jnp.sum(out_cnt)
    return jnp.where(cnt > 0, loss_sum / jnp.maximum(cnt, 1.0),
                     jnp.float32(0.0))
